# baseline (speedup 1.0000x reference)
import functools
import math
from fractions import Fraction

import numpy as np
import jax
import jax.numpy as jnp
from jax import lax
from jax.experimental import pallas as pl
from jax.experimental.pallas import tpu as pltpu
from jax.experimental.pallas import tpu_sc as plsc

_LS1 = [0, 1, 2]
_LS2 = [0, 1, 2]
_DIM1 = sum(2 * l + 1 for l in _LS1)
_DIM2 = sum(2 * l + 1 for l in _LS2)
_ODIM = _DIM1 * _DIM2


def _su2_cg(j1, m1, j2, m2, j3, m3):
    if m3 != m1 + m2:
        return 0.0
    vmin = int(max(-j1 + j2 + m3, -j1 + m1, 0))
    vmax = int(min(j2 + j3 + m1, j3 - j1 + j2, j3 + m3))
    f = math.factorial
    C = ((2.0 * j3 + 1.0) * Fraction(
        f(j3 + j1 - j2) * f(j3 - j1 + j2) * f(j1 + j2 - j3) * f(j3 + m3) * f(j3 - m3),
        f(j1 + j2 + j3 + 1) * f(j1 - m1) * f(j1 + m1) * f(j2 - m2) * f(j2 + m2))) ** 0.5
    S = 0
    for v in range(vmin, vmax + 1):
        S += (-1) ** (v + j2 + m2) * Fraction(
            f(j2 + j3 + m1 - v) * f(j1 - m1 + v),
            f(v) * f(j3 - j1 + j2 - v) * f(j3 + m3 - v) * f(v + j1 - j2 - m3))
    return float(C * S)


def _su2_cg_tensor(j1, j2, j3):
    mat = np.zeros((2 * j1 + 1, 2 * j2 + 1, 2 * j3 + 1))
    for m1 in range(-j1, j1 + 1):
        for m2 in range(-j2, j2 + 1):
            m3 = m1 + m2
            if abs(m3) <= j3:
                mat[j1 + m1, j2 + m2, j3 + m3] = _su2_cg(j1, m1, j2, m2, j3, m3)
    return mat


def _q_real_to_complex(l):
    q = np.zeros((2 * l + 1, 2 * l + 1), dtype=np.complex128)
    for m in range(-l, 0):
        q[l + m, l + abs(m)] = 1 / 2 ** 0.5
        q[l + m, l - abs(m)] = -1j / 2 ** 0.5
    q[l, l] = 1.0
    for m in range(1, l + 1):
        q[l + m, l + abs(m)] = (-1) ** m / 2 ** 0.5
        q[l + m, l - abs(m)] = 1j * (-1) ** m / 2 ** 0.5
    return (-1j) ** l * q


def _wigner_3j(l1, l2, l3):
    Q1 = _q_real_to_complex(l1)
    Q2 = _q_real_to_complex(l2)
    Q3 = _q_real_to_complex(l3)
    C = _su2_cg_tensor(l1, l2, l3).astype(np.complex128)
    C = np.einsum('ij,kl,mn,ikn->jlm', Q1, Q2, np.conj(Q3.T), C)
    C = np.real(C)
    n = np.linalg.norm(C)
    if n > 0:
        C = C / n
    return C


def _build_pair_groups(ls1, ls2):
    lmax2 = max(ls2)
    layout = {}
    idx1 = 0
    for l1 in ls1:
        idx2 = 0
        for l2 in ls2:
            for l3 in range(abs(l1 - l2), l1 + l2 + 1):
                layout.setdefault(l3, []).append((l1, l2, idx1, idx2))
            idx2 += 2 * l2 + 1
        idx1 += 2 * l1 + 1
    blocks = []
    row = 0
    for l3 in sorted(layout.keys()):
        mults = sorted(layout[l3], key=lambda x: x[0] * lmax2 + x[1])
        for (l1, l2, o1, o2) in mults:
            cb = _wigner_3j(l1, l2, l3) * math.sqrt(2 * l3 + 1)
            blocks.append((l1, l2, o1, o2, row, cb))
            row += 2 * l3 + 1
    pair_groups = {}
    for (l1, l2, o1, o2, row, cb) in blocks:
        n1, n2, n3 = cb.shape
        key = (o1, o2, n1, n2)
        klist = pair_groups.setdefault(key, [])
        for c in range(n3):
            bywabs = {}
            for a in range(n1):
                for b in range(n2):
                    w = float(cb[a, b, c])
                    if abs(w) > 1e-14:
                        bywabs.setdefault(round(abs(w), 12), []).append(
                            (a, b, 1.0 if w > 0 else -1.0))
            klist.append((row + c, sorted(bywabs.items())))
    return list(pair_groups.items())


_PAIR_GROUPS = _build_pair_groups(_LS1, _LS2)

_CHUNK = 128
_NWORK = 32


@functools.lru_cache(maxsize=None)
def _make_sc_kernel(batch):
    assert batch % _CHUNK == 0
    nchunk = batch // _CHUNK
    niter = (nchunk + _NWORK - 1) // _NWORK
    ngroup = _CHUNK // 16

    mesh = plsc.VectorSubcoreMesh(core_axis_name="c", subcore_axis_name="s")

    @functools.partial(
        pl.kernel,
        out_type=jax.ShapeDtypeStruct((_ODIM, batch), jnp.float32),
        mesh=mesh,
        scratch_types=[
            pltpu.VMEM((_DIM1, _CHUNK), jnp.float32),
            pltpu.VMEM((_DIM1, _CHUNK), jnp.float32),
            pltpu.VMEM((_DIM2, _CHUNK), jnp.float32),
            pltpu.VMEM((_DIM2, _CHUNK), jnp.float32),
            pltpu.VMEM((_ODIM, _CHUNK), jnp.float32),
            pltpu.VMEM((_ODIM, _CHUNK), jnp.float32),
            pltpu.SemaphoreType.DMA,
            pltpu.SemaphoreType.DMA,
            pltpu.SemaphoreType.DMA,
            pltpu.SemaphoreType.DMA,
            pltpu.SemaphoreType.DMA,
            pltpu.SemaphoreType.DMA,
        ],
        compiler_params=pltpu.CompilerParams(
            needs_layout_passes=False, use_tc_tiling_on_sc=True),
    )
    def sc_kernel(in1_hbm, in2_hbm, out_hbm,
                  x1a, x1b, x2a, x2b, oa, ob,
                  s1a, s1b, s2a, s2b, soa, sob):
        wid = lax.axis_index("s") * 2 + lax.axis_index("c")
        x1 = (x1a, x1b)
        x2 = (x2a, x2b)
        ov = (oa, ob)
        s1 = (s1a, s1b)
        s2 = (s2a, s2b)
        so = (soa, sob)

        def compute_chunk(x1_v, x2_v, o_v):
            def group_body(g, carry):
                off = g * 16
                x1c = [x1_v[i, pl.ds(off, 16)] for i in range(_DIM1)]
                x2c = [x2_v[j, pl.ds(off, 16)] for j in range(_DIM2)]
                for (o1, o2, n1, n2), klist in _PAIR_GROUPS:
                    prods = {}
                    for a in range(n1):
                        for b in range(n2):
                            prods[(a, b)] = x1c[o1 + a] * x2c[o2 + b]
                    for (k, wgroups) in klist:
                        acc = None
                        for (wabs, sterms) in wgroups:
                            a0, b0, sg0 = sterms[0]
                            s = prods[(a0, b0)] if sg0 > 0 else -prods[(a0, b0)]
                            for (a, b, sg) in sterms[1:]:
                                if sg > 0:
                                    s = s + prods[(a, b)]
                                else:
                                    s = s - prods[(a, b)]
                            if wabs != 1.0:
                                s = s * wabs
                            acc = s if acc is None else acc + s
                        o_v[k, pl.ds(off, 16)] = acc
                return carry

            lax.fori_loop(0, ngroup, group_body, 0)

        def start_in(t, p):
            chunk = wid + _NWORK * t

            @pl.when(chunk < nchunk)
            def _():
                r0 = chunk * _CHUNK
                pltpu.async_copy(
                    in1_hbm.at[:, pl.ds(r0, _CHUNK)], x1[p], s1[p])
                pltpu.async_copy(
                    in2_hbm.at[:, pl.ds(r0, _CHUNK)], x2[p], s2[p])

        def stage(t, p):
            chunk = wid + _NWORK * t
            start_in(t + 1, 1 - p)

            @pl.when(chunk < nchunk)
            def _():
                r0 = chunk * _CHUNK
                pltpu.make_async_copy(
                    in1_hbm.at[:, pl.ds(r0, _CHUNK)], x1[p], s1[p]).wait()
                pltpu.make_async_copy(
                    in2_hbm.at[:, pl.ds(r0, _CHUNK)], x2[p], s2[p]).wait()

                @pl.when(t >= 2)
                def _():
                    pltpu.make_async_copy(
                        ov[p], out_hbm.at[:, pl.ds(r0, _CHUNK)], so[p]).wait()

                compute_chunk(x1[p], x2[p], ov[p])
                pltpu.async_copy(
                    ov[p], out_hbm.at[:, pl.ds(r0, _CHUNK)], so[p])

        start_in(0, 0)

        def pair_body(u, carry):
            stage(2 * u, 0)
            stage(2 * u + 1, 1)
            return carry

        lax.fori_loop(0, (niter + 1) // 2, pair_body, 0)

        nactive = (nchunk - wid + _NWORK - 1) // _NWORK

        @pl.when(nactive >= 1)
        def _():
            last = nactive - 1
            r0 = (wid + _NWORK * last) * _CHUNK

            @pl.when(last % 2 == 0)
            def _():
                pltpu.make_async_copy(
                    ov[0], out_hbm.at[:, pl.ds(r0, _CHUNK)], so[0]).wait()

            @pl.when(last % 2 == 1)
            def _():
                pltpu.make_async_copy(
                    ov[1], out_hbm.at[:, pl.ds(r0, _CHUNK)], so[1]).wait()

        @pl.when(nactive >= 2)
        def _():
            prev = nactive - 2
            r0 = (wid + _NWORK * prev) * _CHUNK

            @pl.when(prev % 2 == 0)
            def _():
                pltpu.make_async_copy(
                    ov[0], out_hbm.at[:, pl.ds(r0, _CHUNK)], so[0]).wait()

            @pl.when(prev % 2 == 1)
            def _():
                pltpu.make_async_copy(
                    ov[1], out_hbm.at[:, pl.ds(r0, _CHUNK)], so[1]).wait()

    return sc_kernel


def kernel(in1, in2):
    batch = in1.shape[0]
    out_t = _make_sc_kernel(batch)(in1.T, in2.T)
    return out_t.T


# speedup vs baseline: 2.8428x; 2.8428x over previous
import functools
import math
from fractions import Fraction

import numpy as np
import jax
import jax.numpy as jnp
from jax import lax
from jax.experimental import pallas as pl
from jax.experimental.pallas import tpu as pltpu
from jax.experimental.pallas import tpu_sc as plsc

_LS1 = [0, 1, 2]
_LS2 = [0, 1, 2]
_DIM1 = sum(2 * l + 1 for l in _LS1)
_DIM2 = sum(2 * l + 1 for l in _LS2)
_ODIM = _DIM1 * _DIM2


def _su2_cg(j1, m1, j2, m2, j3, m3):
    if m3 != m1 + m2:
        return 0.0
    vmin = int(max(-j1 + j2 + m3, -j1 + m1, 0))
    vmax = int(min(j2 + j3 + m1, j3 - j1 + j2, j3 + m3))
    f = math.factorial
    C = ((2.0 * j3 + 1.0) * Fraction(
        f(j3 + j1 - j2) * f(j3 - j1 + j2) * f(j1 + j2 - j3) * f(j3 + m3) * f(j3 - m3),
        f(j1 + j2 + j3 + 1) * f(j1 - m1) * f(j1 + m1) * f(j2 - m2) * f(j2 + m2))) ** 0.5
    S = 0
    for v in range(vmin, vmax + 1):
        S += (-1) ** (v + j2 + m2) * Fraction(
            f(j2 + j3 + m1 - v) * f(j1 - m1 + v),
            f(v) * f(j3 - j1 + j2 - v) * f(j3 + m3 - v) * f(v + j1 - j2 - m3))
    return float(C * S)


def _su2_cg_tensor(j1, j2, j3):
    mat = np.zeros((2 * j1 + 1, 2 * j2 + 1, 2 * j3 + 1))
    for m1 in range(-j1, j1 + 1):
        for m2 in range(-j2, j2 + 1):
            m3 = m1 + m2
            if abs(m3) <= j3:
                mat[j1 + m1, j2 + m2, j3 + m3] = _su2_cg(j1, m1, j2, m2, j3, m3)
    return mat


def _q_real_to_complex(l):
    q = np.zeros((2 * l + 1, 2 * l + 1), dtype=np.complex128)
    for m in range(-l, 0):
        q[l + m, l + abs(m)] = 1 / 2 ** 0.5
        q[l + m, l - abs(m)] = -1j / 2 ** 0.5
    q[l, l] = 1.0
    for m in range(1, l + 1):
        q[l + m, l + abs(m)] = (-1) ** m / 2 ** 0.5
        q[l + m, l - abs(m)] = 1j * (-1) ** m / 2 ** 0.5
    return (-1j) ** l * q


def _wigner_3j(l1, l2, l3):
    Q1 = _q_real_to_complex(l1)
    Q2 = _q_real_to_complex(l2)
    Q3 = _q_real_to_complex(l3)
    C = _su2_cg_tensor(l1, l2, l3).astype(np.complex128)
    C = np.einsum('ij,kl,mn,ikn->jlm', Q1, Q2, np.conj(Q3.T), C)
    C = np.real(C)
    n = np.linalg.norm(C)
    if n > 0:
        C = C / n
    return C


def _build_pair_groups(ls1, ls2):
    lmax2 = max(ls2)
    layout = {}
    idx1 = 0
    for l1 in ls1:
        idx2 = 0
        for l2 in ls2:
            for l3 in range(abs(l1 - l2), l1 + l2 + 1):
                layout.setdefault(l3, []).append((l1, l2, idx1, idx2))
            idx2 += 2 * l2 + 1
        idx1 += 2 * l1 + 1
    blocks = []
    row = 0
    for l3 in sorted(layout.keys()):
        mults = sorted(layout[l3], key=lambda x: x[0] * lmax2 + x[1])
        for (l1, l2, o1, o2) in mults:
            cb = _wigner_3j(l1, l2, l3) * math.sqrt(2 * l3 + 1)
            blocks.append((l1, l2, o1, o2, row, cb))
            row += 2 * l3 + 1
    pair_groups = {}
    for (l1, l2, o1, o2, row, cb) in blocks:
        n1, n2, n3 = cb.shape
        key = (o1, o2, n1, n2)
        klist = pair_groups.setdefault(key, [])
        for c in range(n3):
            bywabs = {}
            for a in range(n1):
                for b in range(n2):
                    w = float(cb[a, b, c])
                    if abs(w) > 1e-14:
                        bywabs.setdefault(round(abs(w), 12), []).append(
                            (a, b, 1.0 if w > 0 else -1.0))
            klist.append((row + c, sorted(bywabs.items())))
    return list(pair_groups.items())


_PAIR_GROUPS = _build_pair_groups(_LS1, _LS2)

_CHUNK = 512
_NWORK = 32


@functools.lru_cache(maxsize=None)
def _make_sc_kernel(batch):
    assert batch % 128 == 0 and batch >= _CHUNK
    nchunk = (batch + _CHUNK - 1) // _CHUNK
    niter = (nchunk + _NWORK - 1) // _NWORK
    ngroup = _CHUNK // 16

    mesh = plsc.VectorSubcoreMesh(core_axis_name="c", subcore_axis_name="s")

    @functools.partial(
        pl.kernel,
        out_type=jax.ShapeDtypeStruct((_ODIM, batch), jnp.float32),
        mesh=mesh,
        scratch_types=[
            pltpu.VMEM((_DIM1, _CHUNK), jnp.float32),
            pltpu.VMEM((_DIM1, _CHUNK), jnp.float32),
            pltpu.VMEM((_DIM2, _CHUNK), jnp.float32),
            pltpu.VMEM((_DIM2, _CHUNK), jnp.float32),
            pltpu.VMEM((_ODIM, _CHUNK), jnp.float32),
            pltpu.VMEM((_ODIM, _CHUNK), jnp.float32),
            pltpu.SemaphoreType.DMA,
            pltpu.SemaphoreType.DMA,
            pltpu.SemaphoreType.DMA,
            pltpu.SemaphoreType.DMA,
            pltpu.SemaphoreType.DMA,
            pltpu.SemaphoreType.DMA,
        ],
        compiler_params=pltpu.CompilerParams(
            needs_layout_passes=False, use_tc_tiling_on_sc=True),
    )
    def sc_kernel(in1_hbm, in2_hbm, out_hbm,
                  x1a, x1b, x2a, x2b, oa, ob,
                  s1a, s1b, s2a, s2b, soa, sob):
        wid = lax.axis_index("s") * 2 + lax.axis_index("c")
        x1 = (x1a, x1b)
        x2 = (x2a, x2b)
        ov = (oa, ob)
        s1 = (s1a, s1b)
        s2 = (s2a, s2b)
        so = (soa, sob)

        def compute_chunk(x1_v, x2_v, o_v):
            def group_body(g, carry):
                off = g * 16
                x1c = [x1_v[i, pl.ds(off, 16)] for i in range(_DIM1)]
                x2c = [x2_v[j, pl.ds(off, 16)] for j in range(_DIM2)]
                for (o1, o2, n1, n2), klist in _PAIR_GROUPS:
                    prods = {}
                    for a in range(n1):
                        for b in range(n2):
                            prods[(a, b)] = x1c[o1 + a] * x2c[o2 + b]
                    for (k, wgroups) in klist:
                        acc = None
                        for (wabs, sterms) in wgroups:
                            a0, b0, sg0 = sterms[0]
                            s = prods[(a0, b0)] if sg0 > 0 else -prods[(a0, b0)]
                            for (a, b, sg) in sterms[1:]:
                                if sg > 0:
                                    s = s + prods[(a, b)]
                                else:
                                    s = s - prods[(a, b)]
                            if wabs != 1.0:
                                s = s * wabs
                            acc = s if acc is None else acc + s
                        o_v[k, pl.ds(off, 16)] = acc
                return carry

            lax.fori_loop(0, ngroup, group_body, 0)

        def start_in(t, p):
            chunk = wid + _NWORK * t

            @pl.when(chunk < nchunk)
            def _():
                r0 = jnp.minimum(chunk * _CHUNK, batch - _CHUNK)
                pltpu.async_copy(
                    in1_hbm.at[:, pl.ds(r0, _CHUNK)], x1[p], s1[p])
                pltpu.async_copy(
                    in2_hbm.at[:, pl.ds(r0, _CHUNK)], x2[p], s2[p])

        def stage(t, p):
            chunk = wid + _NWORK * t
            start_in(t + 1, 1 - p)

            @pl.when(chunk < nchunk)
            def _():
                r0 = jnp.minimum(chunk * _CHUNK, batch - _CHUNK)
                pltpu.make_async_copy(
                    in1_hbm.at[:, pl.ds(r0, _CHUNK)], x1[p], s1[p]).wait()
                pltpu.make_async_copy(
                    in2_hbm.at[:, pl.ds(r0, _CHUNK)], x2[p], s2[p]).wait()

                @pl.when(t >= 2)
                def _():
                    pltpu.make_async_copy(
                        ov[p], out_hbm.at[:, pl.ds(r0, _CHUNK)], so[p]).wait()

                compute_chunk(x1[p], x2[p], ov[p])
                pltpu.async_copy(
                    ov[p], out_hbm.at[:, pl.ds(r0, _CHUNK)], so[p])

        start_in(0, 0)

        def pair_body(u, carry):
            stage(2 * u, 0)
            stage(2 * u + 1, 1)
            return carry

        lax.fori_loop(0, (niter + 1) // 2, pair_body, 0)

        nactive = (nchunk - wid + _NWORK - 1) // _NWORK

        @pl.when(nactive >= 1)
        def _():
            last = nactive - 1
            r0 = jnp.minimum((wid + _NWORK * last) * _CHUNK, batch - _CHUNK)

            @pl.when(last % 2 == 0)
            def _():
                pltpu.make_async_copy(
                    ov[0], out_hbm.at[:, pl.ds(r0, _CHUNK)], so[0]).wait()

            @pl.when(last % 2 == 1)
            def _():
                pltpu.make_async_copy(
                    ov[1], out_hbm.at[:, pl.ds(r0, _CHUNK)], so[1]).wait()

        @pl.when(nactive >= 2)
        def _():
            prev = nactive - 2
            r0 = jnp.minimum((wid + _NWORK * prev) * _CHUNK, batch - _CHUNK)

            @pl.when(prev % 2 == 0)
            def _():
                pltpu.make_async_copy(
                    ov[0], out_hbm.at[:, pl.ds(r0, _CHUNK)], so[0]).wait()

            @pl.when(prev % 2 == 1)
            def _():
                pltpu.make_async_copy(
                    ov[1], out_hbm.at[:, pl.ds(r0, _CHUNK)], so[1]).wait()

    return sc_kernel


def kernel(in1, in2):
    batch = in1.shape[0]
    out_t = _make_sc_kernel(batch)(in1.T, in2.T)
    return out_t.T
